# Initial kernel scaffold; baseline (speedup 1.0000x reference)
import jax
import jax.numpy as jnp
from jax.experimental import pallas as pl


def kernel(x, sadj, fadj, W1, b1, W2, b2, W3, b3, Wg1a, bg1a, Wg2a, bg2a, Wg1b, bg1b, Wg2b, bg2b, M1, M2, bM2, M3, bM3, logit_scale):
    raise NotImplementedError("write your pallas kernel here")


# speedup vs baseline: 1.6133x; 1.6133x over previous
import jax
import jax.numpy as jnp
from jax.experimental import pallas as pl
from jax.experimental.pallas import tpu as pltpu

N = 4096
F32 = jnp.float32
BLK = 256
BLK_E = 256
BLK_S = 512


def _dot(a, b):
    return jnp.dot(a, b, preferred_element_type=F32)


def _xw_kernel(x_ref, w_ref, o_ref):
    o_ref[...] = _dot(x_ref[...], w_ref[...])


def _ab_kernel(s_ref, f_ref, xw_ref, b_ref, sh1_ref, t1_ref, fh1_ref, t2_ref):
    xw = xw_ref[...]
    b = b_ref[...]
    pa = _dot(s_ref[...], xw[:, :512])
    sh1_ref[...] = jax.nn.relu(pa[:, :256] + b[:, :256])
    t1_ref[...] = jax.nn.relu(pa[:, 256:512] + b[:, 256:512])
    pb1 = _dot(f_ref[...], xw[:, :256])
    fh1_ref[...] = jax.nn.relu(pb1 + b[:, :256])
    pb2 = _dot(f_ref[...], xw[:, 512:768])
    t2_ref[...] = jax.nn.relu(pb2 + b[:, 512:768])


def _mid_kernel(sh1_ref, fh1_ref, t1_ref, t2_ref, w23_ref, wg2a_ref,
                wg2b_ref, r1_ref, r2_ref):
    w23 = w23_ref[...]
    r1_ref[:, :256] = _dot(sh1_ref[...], w23)
    r1_ref[:, 256:512] = _dot(fh1_ref[...], w23)
    r1_ref[:, 512:640] = _dot(t1_ref[...], wg2a_ref[...])
    r2_ref[...] = _dot(t2_ref[...], wg2b_ref[...])


def _cd_kernel(s_ref, f_ref, r1_ref, r2_ref, bc_ref, bg_ref,
               smu_ref, slv_ref, fmu_ref, flv_ref, h1_ref, h2_ref):
    p = jax.nn.relu(_dot(s_ref[...], r1_ref[...]) + bc_ref[...])
    smu_ref[...] = p[:, 0:128]
    slv_ref[...] = p[:, 128:256]
    fmu_ref[...] = p[:, 256:384]
    flv_ref[...] = p[:, 384:512]
    h1_ref[...] = p[:, 512:640]
    h2_ref[...] = jax.nn.relu(_dot(f_ref[...], r2_ref[...]) + bg_ref[...])


def _e_kernel(h1_ref, h2_ref, h1t_ref, h2t_ref, m1_ref, m2_ref, bm2_ref,
              m3_ref, bm3_ref, ls_ref,
              a1_ref, a2_ref, e1_ref, e2_ref, out_ref, els_ref):
    r1 = h1_ref[...]
    r2 = h2_ref[...]
    a1_ref[...] = jax.nn.sigmoid(_dot(r1, h1t_ref[...]))
    a2_ref[...] = jax.nn.sigmoid(_dot(r2, h2t_ref[...]))
    n1 = jnp.sqrt(jnp.sum(r1 * r1, axis=1, keepdims=True))
    n2 = jnp.sqrt(jnp.sum(r2 * r2, axis=1, keepdims=True))
    e1_ref[...] = r1 / n1
    e2_ref[...] = r2 / n2
    z = jnp.concatenate([r1, r2], axis=1)
    t = _dot(z, m1_ref[...])
    t = _dot(t, m2_ref[...]) + bm2_ref[...]
    t = _dot(t, m3_ref[...]) + bm3_ref[...]
    m = jnp.max(t, axis=1, keepdims=True)
    out_ref[...] = t - m - jnp.log(jnp.sum(jnp.exp(t - m), axis=1,
                                           keepdims=True))
    els_ref[...] = jnp.exp(ls_ref[...])


def _cparams():
    return pltpu.CompilerParams(dimension_semantics=("parallel",))


def kernel(x, sadj, fadj, W1, b1, W2, b2, W3, b3, Wg1a, bg1a, Wg2a, bg2a,
           Wg1b, bg1b, Wg2b, bg2b, M1, M2, bM2, M3, bM3, logit_scale):
    Wc = jnp.concatenate([W1, Wg1a, Wg1b], axis=1)
    bab = jnp.concatenate([b1, bg1a, bg1b]).reshape(1, 768)

    XW = pl.pallas_call(
        _xw_kernel,
        grid=(N // BLK_S,),
        in_specs=[
            pl.BlockSpec((BLK_S, 512), lambda i: (i, 0)),
            pl.BlockSpec((512, 768), lambda i: (0, 0)),
        ],
        out_specs=pl.BlockSpec((BLK_S, 768), lambda i: (i, 0)),
        out_shape=jax.ShapeDtypeStruct((N, 768), F32),
        compiler_params=_cparams(),
    )(x, Wc)

    adj_spec = pl.BlockSpec((BLK, N), lambda i: (i, 0))
    res = lambda shape: pl.BlockSpec(shape, lambda i: (0, 0))
    h256 = pl.BlockSpec((BLK, 256), lambda i: (i, 0))
    sh1, t1, fh1, t2 = pl.pallas_call(
        _ab_kernel,
        grid=(N // BLK,),
        in_specs=[adj_spec, adj_spec, res((N, 768)), res((1, 768))],
        out_specs=[h256, h256, h256, h256],
        out_shape=[jax.ShapeDtypeStruct((N, 256), F32)] * 4,
        compiler_params=_cparams(),
    )(sadj, fadj, XW, bab)

    w23 = jnp.concatenate([W2, W3], axis=1)
    s256 = pl.BlockSpec((BLK_S, 256), lambda i: (i, 0))
    R1, R2 = pl.pallas_call(
        _mid_kernel,
        grid=(N // BLK_S,),
        in_specs=[s256, s256, s256, s256, res((256, 256)),
                  res((256, 128)), res((256, 128))],
        out_specs=[pl.BlockSpec((BLK_S, 640), lambda i: (i, 0)),
                   pl.BlockSpec((BLK_S, 128), lambda i: (i, 0))],
        out_shape=[jax.ShapeDtypeStruct((N, 640), F32),
                   jax.ShapeDtypeStruct((N, 128), F32)],
        compiler_params=_cparams(),
    )(sh1, fh1, t1, t2, w23, Wg2a, Wg2b)

    bc = jnp.concatenate([b2, b3, b2, b3, bg2a]).reshape(1, 640)
    bg = bg2b.reshape(1, 128)
    h128 = pl.BlockSpec((BLK, 128), lambda i: (i, 0))
    smu, slv, fmu, flv, h1, h2 = pl.pallas_call(
        _cd_kernel,
        grid=(N // BLK,),
        in_specs=[adj_spec, adj_spec, res((N, 640)), res((N, 128)),
                  res((1, 640)), res((1, 128))],
        out_specs=[h128] * 6,
        out_shape=[jax.ShapeDtypeStruct((N, 128), F32)] * 6,
        compiler_params=_cparams(),
    )(sadj, fadj, R1, R2, bc, bg)

    h1t = h1.T
    h2t = h2.T
    he = pl.BlockSpec((BLK_E, 128), lambda i: (i, 0))
    A1, A2, emb1, emb2, out, els = pl.pallas_call(
        _e_kernel,
        grid=(N // BLK_E,),
        in_specs=[he, he, res((128, N)), res((128, N)), res((256, 256)),
                  res((256, 128)), res((1, 128)), res((128, 16)),
                  res((1, 16)), res((1, 1))],
        out_specs=[pl.BlockSpec((BLK_E, N), lambda i: (i, 0)),
                   pl.BlockSpec((BLK_E, N), lambda i: (i, 0)),
                   he, he,
                   pl.BlockSpec((BLK_E, 16), lambda i: (i, 0)),
                   pl.BlockSpec((1, 1), lambda i: (0, 0))],
        out_shape=[jax.ShapeDtypeStruct((N, N), F32),
                   jax.ShapeDtypeStruct((N, N), F32),
                   jax.ShapeDtypeStruct((N, 128), F32),
                   jax.ShapeDtypeStruct((N, 128), F32),
                   jax.ShapeDtypeStruct((N, 16), F32),
                   jax.ShapeDtypeStruct((1, 1), F32)],
        compiler_params=_cparams(),
    )(h1, h2, h1t, h2t, M1, M2, bM2.reshape(1, 128), M3,
      bM3.reshape(1, 16), logit_scale.reshape(1, 1))

    return (out, A1, A2, emb1, emb2, els.reshape(()), smu, slv, fmu, flv)


# speedup vs baseline: 1.7286x; 1.0714x over previous
import jax
import jax.numpy as jnp
from jax.experimental import pallas as pl
from jax.experimental.pallas import tpu as pltpu

N = 4096
F32 = jnp.float32
BF16 = jnp.bfloat16
BLK = 256
BLK_E = 256
BLK_S = 512


def _dot(a, b):
    return jnp.dot(a.astype(BF16), b.astype(BF16),
                   preferred_element_type=F32)


def _xw_kernel(x_ref, w_ref, o_ref):
    o_ref[...] = _dot(x_ref[...], w_ref[...]).astype(BF16)


def _ab_kernel(s_ref, f_ref, xw_ref, b_ref, sh1_ref, t1_ref, fh1_ref, t2_ref):
    xw = xw_ref[...]
    b = b_ref[...]
    pa = _dot(s_ref[...], xw[:, :512])
    sh1_ref[...] = jax.nn.relu(pa[:, :256] + b[:, :256]).astype(BF16)
    t1_ref[...] = jax.nn.relu(pa[:, 256:512] + b[:, 256:512]).astype(BF16)
    pb1 = _dot(f_ref[...], xw[:, :256])
    fh1_ref[...] = jax.nn.relu(pb1 + b[:, :256]).astype(BF16)
    pb2 = _dot(f_ref[...], xw[:, 512:768])
    t2_ref[...] = jax.nn.relu(pb2 + b[:, 512:768]).astype(BF16)


def _mid_kernel(sh1_ref, fh1_ref, t1_ref, t2_ref, w23_ref, wg2a_ref,
                wg2b_ref, r1_ref, r2_ref):
    w23 = w23_ref[...]
    r1_ref[:, :256] = _dot(sh1_ref[...], w23).astype(BF16)
    r1_ref[:, 256:512] = _dot(fh1_ref[...], w23).astype(BF16)
    r1_ref[:, 512:640] = _dot(t1_ref[...], wg2a_ref[...]).astype(BF16)
    r2_ref[...] = _dot(t2_ref[...], wg2b_ref[...]).astype(BF16)


def _cd_kernel(s_ref, f_ref, r1_ref, r2_ref, bc_ref, bg_ref,
               smu_ref, slv_ref, fmu_ref, flv_ref, h1_ref, h2_ref):
    p = jax.nn.relu(_dot(s_ref[...], r1_ref[...]) + bc_ref[...])
    smu_ref[...] = p[:, 0:128]
    slv_ref[...] = p[:, 128:256]
    fmu_ref[...] = p[:, 256:384]
    flv_ref[...] = p[:, 384:512]
    h1_ref[...] = p[:, 512:640]
    h2_ref[...] = jax.nn.relu(_dot(f_ref[...], r2_ref[...]) + bg_ref[...])


def _e_kernel(h1_ref, h2_ref, h1t_ref, h2t_ref, m1_ref, m2_ref, bm2_ref,
              m3_ref, bm3_ref, ls_ref,
              a1_ref, a2_ref, e1_ref, e2_ref, out_ref, els_ref):
    r1 = h1_ref[...]
    r2 = h2_ref[...]
    a1_ref[...] = jax.nn.sigmoid(_dot(r1, h1t_ref[...]))
    a2_ref[...] = jax.nn.sigmoid(_dot(r2, h2t_ref[...]))
    n1 = jnp.sqrt(jnp.sum(r1 * r1, axis=1, keepdims=True))
    n2 = jnp.sqrt(jnp.sum(r2 * r2, axis=1, keepdims=True))
    e1_ref[...] = r1 / n1
    e2_ref[...] = r2 / n2
    z = jnp.concatenate([r1, r2], axis=1)
    t = _dot(z, m1_ref[...])
    t = _dot(t, m2_ref[...]) + bm2_ref[...]
    t = _dot(t, m3_ref[...]) + bm3_ref[...]
    m = jnp.max(t, axis=1, keepdims=True)
    out_ref[...] = t - m - jnp.log(jnp.sum(jnp.exp(t - m), axis=1,
                                           keepdims=True))
    els_ref[...] = jnp.exp(ls_ref[...])


def _cparams():
    return pltpu.CompilerParams(dimension_semantics=("parallel",))


def kernel(x, sadj, fadj, W1, b1, W2, b2, W3, b3, Wg1a, bg1a, Wg2a, bg2a,
           Wg1b, bg1b, Wg2b, bg2b, M1, M2, bM2, M3, bM3, logit_scale):
    Wc = jnp.concatenate([W1, Wg1a, Wg1b], axis=1)
    bab = jnp.concatenate([b1, bg1a, bg1b]).reshape(1, 768)

    XW = pl.pallas_call(
        _xw_kernel,
        grid=(N // BLK_S,),
        in_specs=[
            pl.BlockSpec((BLK_S, 512), lambda i: (i, 0)),
            pl.BlockSpec((512, 768), lambda i: (0, 0)),
        ],
        out_specs=pl.BlockSpec((BLK_S, 768), lambda i: (i, 0)),
        out_shape=jax.ShapeDtypeStruct((N, 768), BF16),
        compiler_params=_cparams(),
    )(x, Wc)

    adj_spec = pl.BlockSpec((BLK, N), lambda i: (i, 0))
    res = lambda shape: pl.BlockSpec(shape, lambda i: (0, 0))
    h256 = pl.BlockSpec((BLK, 256), lambda i: (i, 0))
    sh1, t1, fh1, t2 = pl.pallas_call(
        _ab_kernel,
        grid=(N // BLK,),
        in_specs=[adj_spec, adj_spec, res((N, 768)), res((1, 768))],
        out_specs=[h256, h256, h256, h256],
        out_shape=[jax.ShapeDtypeStruct((N, 256), BF16)] * 4,
        compiler_params=_cparams(),
    )(sadj, fadj, XW, bab)

    w23 = jnp.concatenate([W2, W3], axis=1)
    s256 = pl.BlockSpec((BLK_S, 256), lambda i: (i, 0))
    R1, R2 = pl.pallas_call(
        _mid_kernel,
        grid=(N // BLK_S,),
        in_specs=[s256, s256, s256, s256, res((256, 256)),
                  res((256, 128)), res((256, 128))],
        out_specs=[pl.BlockSpec((BLK_S, 640), lambda i: (i, 0)),
                   pl.BlockSpec((BLK_S, 128), lambda i: (i, 0))],
        out_shape=[jax.ShapeDtypeStruct((N, 640), BF16),
                   jax.ShapeDtypeStruct((N, 128), BF16)],
        compiler_params=_cparams(),
    )(sh1, fh1, t1, t2, w23, Wg2a, Wg2b)

    bc = jnp.concatenate([b2, b3, b2, b3, bg2a]).reshape(1, 640)
    bg = bg2b.reshape(1, 128)
    h128 = pl.BlockSpec((BLK, 128), lambda i: (i, 0))
    smu, slv, fmu, flv, h1, h2 = pl.pallas_call(
        _cd_kernel,
        grid=(N // BLK,),
        in_specs=[adj_spec, adj_spec, res((N, 640)), res((N, 128)),
                  res((1, 640)), res((1, 128))],
        out_specs=[h128] * 6,
        out_shape=[jax.ShapeDtypeStruct((N, 128), F32)] * 6,
        compiler_params=_cparams(),
    )(sadj, fadj, R1, R2, bc, bg)

    h1t = h1.astype(BF16).T
    h2t = h2.astype(BF16).T
    he = pl.BlockSpec((BLK_E, 128), lambda i: (i, 0))
    A1, A2, emb1, emb2, out, els = pl.pallas_call(
        _e_kernel,
        grid=(N // BLK_E,),
        in_specs=[he, he, res((128, N)), res((128, N)), res((256, 256)),
                  res((256, 128)), res((1, 128)), res((128, 16)),
                  res((1, 16)), res((1, 1))],
        out_specs=[pl.BlockSpec((BLK_E, N), lambda i: (i, 0)),
                   pl.BlockSpec((BLK_E, N), lambda i: (i, 0)),
                   he, he,
                   pl.BlockSpec((BLK_E, 16), lambda i: (i, 0)),
                   pl.BlockSpec((1, 1), lambda i: (0, 0))],
        out_shape=[jax.ShapeDtypeStruct((N, N), F32),
                   jax.ShapeDtypeStruct((N, N), F32),
                   jax.ShapeDtypeStruct((N, 128), F32),
                   jax.ShapeDtypeStruct((N, 128), F32),
                   jax.ShapeDtypeStruct((N, 16), F32),
                   jax.ShapeDtypeStruct((1, 1), F32)],
        compiler_params=_cparams(),
    )(h1, h2, h1t, h2t, M1, M2, bM2.reshape(1, 128), M3,
      bM3.reshape(1, 16), logit_scale.reshape(1, 1))

    return (out, A1, A2, emb1, emb2, els.reshape(()), smu, slv, fmu, flv)


# speedup vs baseline: 1.7401x; 1.0067x over previous
import jax
import jax.numpy as jnp
from jax.experimental import pallas as pl
from jax.experimental.pallas import tpu as pltpu

N = 4096
F32 = jnp.float32
BF16 = jnp.bfloat16
BLK = 256
NB = N // BLK
BLK_S = 512


def _dot(a, b):
    return jnp.dot(a.astype(BF16), b.astype(BF16),
                   preferred_element_type=F32)


def _xw_kernel(x_ref, w_ref, os_ref, of_ref):
    xw = _dot(x_ref[...], w_ref[...]).astype(BF16)
    os_ref[...] = xw[:, :512]
    of_ref[:, :256] = xw[:, :256]
    of_ref[:, 256:512] = xw[:, 512:768]


def _kf_kernel(f_ref, xwf_ref, babf_ref, wg2b_ref, bg_ref,
               fh1_ref, h2_ref, h2t_ref,
               fadj_bf, r2s):
    g = pl.program_id(0)
    i = pl.program_id(1)
    rows = pl.ds(i * BLK, BLK)

    @pl.when(g == 0)
    def _phase0():
        fb = f_ref[...].astype(BF16)
        fadj_bf[rows, :] = fb
        xwf = xwf_ref[...]
        b = babf_ref[...]
        fh1 = jax.nn.relu(_dot(fb, xwf[:, :256]) + b[:, :256])
        t2 = jax.nn.relu(_dot(fb, xwf[:, 256:512]) + b[:, 256:512])
        fh1_ref[...] = fh1.astype(BF16)
        r2s[rows, :] = _dot(t2.astype(BF16), wg2b_ref[...]).astype(BF16)

    @pl.when(g == 1)
    def _phase1():
        fb = fadj_bf[rows, :]
        h2b = jax.nn.relu(_dot(fb, r2s[...]) + bg_ref[...])
        h2_ref[...] = h2b
        h2t_ref[...] = h2b.T.astype(BF16)


def _ks_kernel(s_ref, xws_ref, babs_ref, fh1_ref, w23_ref, wg2a_ref, bc_ref,
               smu_ref, slv_ref, fmu_ref, flv_ref, h1_ref, h1t_ref,
               sadj_bf, r1s):
    g = pl.program_id(0)
    i = pl.program_id(1)
    rows = pl.ds(i * BLK, BLK)

    @pl.when(g == 0)
    def _phase0():
        sb = s_ref[...].astype(BF16)
        sadj_bf[rows, :] = sb
        xws = xws_ref[...]
        b = babs_ref[...]
        pa = _dot(sb, xws)
        sh1 = jax.nn.relu(pa[:, :256] + b[:, :256]).astype(BF16)
        t1 = jax.nn.relu(pa[:, 256:512] + b[:, 256:512]).astype(BF16)
        w23 = w23_ref[...]
        r1s[rows, 0:256] = _dot(sh1, w23).astype(BF16)
        r1s[rows, 256:512] = _dot(fh1_ref[rows, :], w23).astype(BF16)
        r1s[rows, 512:640] = _dot(t1, wg2a_ref[...]).astype(BF16)

    @pl.when(g == 1)
    def _phase1():
        sb = sadj_bf[rows, :]
        p = jax.nn.relu(_dot(sb, r1s[...]) + bc_ref[...])
        smu_ref[...] = p[:, 0:128]
        slv_ref[...] = p[:, 128:256]
        fmu_ref[...] = p[:, 256:384]
        flv_ref[...] = p[:, 384:512]
        h1b = p[:, 512:640]
        h1_ref[...] = h1b
        h1t_ref[...] = h1b.T.astype(BF16)


def _ke_kernel(h1_ref, h2_ref, h1t_ref, h2t_ref, m1_ref, m2_ref, bm2_ref,
               m3_ref, bm3_ref, ls_ref,
               a1_ref, a2_ref, e1_ref, e2_ref, out_ref, els_ref):
    r1 = h1_ref[...]
    r2 = h2_ref[...]
    a1_ref[...] = jax.nn.sigmoid(_dot(r1, h1t_ref[...]))
    a2_ref[...] = jax.nn.sigmoid(_dot(r2, h2t_ref[...]))
    n1 = jnp.sqrt(jnp.sum(r1 * r1, axis=1, keepdims=True))
    n2 = jnp.sqrt(jnp.sum(r2 * r2, axis=1, keepdims=True))
    e1_ref[...] = r1 / n1
    e2_ref[...] = r2 / n2
    z = jnp.concatenate([r1, r2], axis=1)
    t = _dot(z, m1_ref[...])
    t = _dot(t, m2_ref[...]) + bm2_ref[...]
    t = _dot(t, m3_ref[...]) + bm3_ref[...]
    m = jnp.max(t, axis=1, keepdims=True)
    out_ref[...] = t - m - jnp.log(jnp.sum(jnp.exp(t - m), axis=1,
                                           keepdims=True))
    els_ref[...] = jnp.exp(ls_ref[...])


def kernel(x, sadj, fadj, W1, b1, W2, b2, W3, b3, Wg1a, bg1a, Wg2a, bg2a,
           Wg1b, bg1b, Wg2b, bg2b, M1, M2, bM2, M3, bM3, logit_scale):
    Wc = jnp.concatenate([W1, Wg1a, Wg1b], axis=1)
    XWs, XWf = pl.pallas_call(
        _xw_kernel,
        grid=(N // BLK_S,),
        in_specs=[
            pl.BlockSpec((BLK_S, 512), lambda i: (i, 0)),
            pl.BlockSpec((512, 768), lambda i: (0, 0)),
        ],
        out_specs=[pl.BlockSpec((BLK_S, 512), lambda i: (i, 0)),
                   pl.BlockSpec((BLK_S, 512), lambda i: (i, 0))],
        out_shape=[jax.ShapeDtypeStruct((N, 512), BF16),
                   jax.ShapeDtypeStruct((N, 512), BF16)],
        compiler_params=pltpu.CompilerParams(
            dimension_semantics=("parallel",)),
    )(x, Wc)

    last = NB - 1
    adj_spec = pl.BlockSpec((BLK, N),
                            lambda g, i: (jnp.where(g == 0, i, last), 0))
    res2 = lambda shape: pl.BlockSpec(shape, lambda g, i: (0, 0))
    p0b = lambda w: pl.BlockSpec((BLK, w),
                                 lambda g, i: (jnp.where(g == 0, i, last), 0))
    p1b = lambda w: pl.BlockSpec((BLK, w),
                                 lambda g, i: (jnp.where(g == 1, i, 0), 0))
    p1t = pl.BlockSpec((128, BLK),
                       lambda g, i: (0, jnp.where(g == 1, i, 0)))
    arb2 = pltpu.CompilerParams(
        dimension_semantics=("arbitrary", "arbitrary"))

    babf = jnp.concatenate([b1, bg1b]).reshape(1, 512)
    fh1, h2, h2t = pl.pallas_call(
        _kf_kernel,
        grid=(2, NB),
        in_specs=[adj_spec, res2((N, 512)), res2((1, 512)),
                  res2((256, 128)), res2((1, 128))],
        out_specs=[p0b(256), p1b(128), p1t],
        out_shape=[jax.ShapeDtypeStruct((N, 256), BF16),
                   jax.ShapeDtypeStruct((N, 128), F32),
                   jax.ShapeDtypeStruct((128, N), BF16)],
        scratch_shapes=[pltpu.VMEM((N, N), BF16),
                        pltpu.VMEM((N, 128), BF16)],
        compiler_params=arb2,
    )(fadj, XWf, babf, Wg2b.astype(BF16), bg2b.reshape(1, 128))

    babs = jnp.concatenate([b1, bg1a]).reshape(1, 512)
    w23 = jnp.concatenate([W2, W3], axis=1).astype(BF16)
    bc = jnp.concatenate([b2, b3, b2, b3, bg2a]).reshape(1, 640)
    smu, slv, fmu, flv, h1, h1t = pl.pallas_call(
        _ks_kernel,
        grid=(2, NB),
        in_specs=[adj_spec, res2((N, 512)), res2((1, 512)), res2((N, 256)),
                  res2((256, 256)), res2((256, 128)), res2((1, 640))],
        out_specs=[p1b(128), p1b(128), p1b(128), p1b(128), p1b(128), p1t],
        out_shape=[jax.ShapeDtypeStruct((N, 128), F32)] * 5 +
                  [jax.ShapeDtypeStruct((128, N), BF16)],
        scratch_shapes=[pltpu.VMEM((N, N), BF16),
                        pltpu.VMEM((N, 640), BF16)],
        compiler_params=arb2,
    )(sadj, XWs, babs, fh1, w23, Wg2a.astype(BF16), bc)

    he = pl.BlockSpec((BLK, 128), lambda i: (i, 0))
    res = lambda shape: pl.BlockSpec(shape, lambda i: (0, 0))
    A1, A2, emb1, emb2, out, els = pl.pallas_call(
        _ke_kernel,
        grid=(NB,),
        in_specs=[he, he, res((128, N)), res((128, N)), res((256, 256)),
                  res((256, 128)), res((1, 128)), res((128, 16)),
                  res((1, 16)), res((1, 1))],
        out_specs=[pl.BlockSpec((BLK, N), lambda i: (i, 0)),
                   pl.BlockSpec((BLK, N), lambda i: (i, 0)),
                   he, he,
                   pl.BlockSpec((BLK, 16), lambda i: (i, 0)),
                   pl.BlockSpec((1, 1), lambda i: (0, 0))],
        out_shape=[jax.ShapeDtypeStruct((N, N), F32),
                   jax.ShapeDtypeStruct((N, N), F32),
                   jax.ShapeDtypeStruct((N, 128), F32),
                   jax.ShapeDtypeStruct((N, 128), F32),
                   jax.ShapeDtypeStruct((N, 16), F32),
                   jax.ShapeDtypeStruct((1, 1), F32)],
        compiler_params=pltpu.CompilerParams(
            dimension_semantics=("parallel",)),
    )(h1, h2, h1t, h2t, M1, M2, bM2.reshape(1, 128), M3,
      bM3.reshape(1, 16), logit_scale.reshape(1, 1))

    return (out, A1, A2, emb1, emb2, els.reshape(()), smu, slv, fmu, flv)
